# baseline (speedup 1.0000x reference)
import functools

import numpy as np
import jax
import jax.numpy as jnp
from jax import lax
from jax.experimental import pallas as pl
from jax.experimental.pallas import tpu as pltpu
from jax.experimental.pallas import tpu_sc as plsc

_FPN_ANCHOR_SCALES = [32, 64, 128, 256, 512]
_FPN_FEAT_STRIDES = [4, 8, 16, 32, 64]
_ANCHOR_RATIOS = [0.5, 1.0, 2.0]
_IM_SIZE = 512
_PRE = 6000
_POST = 300
_TH = 0.7
_LANES = 128
_NB = 48
_PAD_N = _NB * _LANES
_HIGH = lax.Precision.HIGHEST


def _gen_anchors() -> np.ndarray:
    all_boxes = []
    ratios = np.array(_ANCHOR_RATIOS, dtype=np.float64)
    for scale, stride in zip(_FPN_ANCHOR_SCALES, _FPN_FEAT_STRIDES):
        fh = _IM_SIZE // stride
        fw = _IM_SIZE // stride
        heights = scale / np.sqrt(ratios)
        widths = scale * np.sqrt(ratios)
        shifts_y = np.arange(0, fh) * stride
        shifts_x = np.arange(0, fw) * stride
        sx, sy = np.meshgrid(shifts_x, shifts_y)
        box_w, box_cx = np.meshgrid(widths, sx.flatten())
        box_h, box_cy = np.meshgrid(heights, sy.flatten())
        boxes = np.stack([box_cx - 0.5 * box_w, box_cy - 0.5 * box_h,
                          box_cx + 0.5 * box_w, box_cy + 0.5 * box_h],
                         axis=2).reshape(-1, 4)
        all_boxes.append(boxes)
    return np.concatenate(all_boxes, axis=0).astype(np.float32)


_ANCHORS = _gen_anchors()


def _tr(x):
    eye = jnp.eye(x.shape[0], dtype=jnp.float32)
    return lax.dot_general(x, eye, (((0,), (0,)), ((), ())), precision=_HIGH)


def _nms_body(nblk, valid_n, im_ref, anch_ref, dl_ref, dr_ref,
              out_l_ref, out_r_ref, out_cnt_ref,
              bs_l, bs_r, kp_l, kp_r, acc_l, acc_r, cnt):
    i = pl.program_id(0)
    cnt[0] = 0
    acc_l[...] = jnp.zeros(acc_l.shape, jnp.float32)
    acc_r[...] = jnp.zeros(acc_r.shape, jnp.float32)
    imx = im_ref[0, 0:1, :]
    imy = im_ref[0, 1:2, :]

    iota_r = lax.broadcasted_iota(jnp.int32, (_LANES, _LANES), 0)
    iota_c = lax.broadcasted_iota(jnp.int32, (_LANES, _LANES), 1)
    lt_strict = jnp.where(iota_r > iota_c, 1.0, 0.0).astype(jnp.float32)
    sub_iota = lax.broadcasted_iota(jnp.int32, (_LANES, 1), 0)
    q_iota = lax.broadcasted_iota(
        jnp.int32, (3 * _LANES, _LANES), 0).astype(jnp.float32)

    def decode_side(d_ref, bs_ref, k):
        a = anch_ref[0, k]
        d = d_ref[0, k]
        x1a, y1a, x2a, y2a = a[1:2], a[2:3], a[3:4], a[4:5]
        dx, dy, dw, dh = d[1:2], d[2:3], d[3:4], d[4:5]
        w = x2a - x1a + 1.0
        h = y2a - y1a + 1.0
        cx = x1a + 0.5 * w
        cy = y1a + 0.5 * h
        pcx = dx * w + cx
        pcy = dy * h + cy
        pw = jnp.exp(dw) * w
        ph = jnp.exp(dh) * h
        px1 = jnp.clip(pcx - 0.5 * pw, 0.0, imx)
        py1 = jnp.clip(pcy - 0.5 * ph, 0.0, imy)
        px2 = jnp.clip(pcx + 0.5 * pw, 0.0, imx)
        py2 = jnp.clip(pcy + 0.5 * ph, 0.0, imy)
        rows = jnp.concatenate(
            [jnp.zeros((1, _LANES), jnp.float32), px1, py1, px2, py2,
             jnp.zeros((3, _LANES), jnp.float32)], axis=0)
        bs_ref[pl.ds(k, 1)] = rows.reshape(1, 8, _LANES)
        return rows

    def side_keep(rows, bs_ref, kp_ref, k, avail0):
        bT = _tr(rows)
        x1c, y1c = bT[:, 1:2], bT[:, 2:3]
        x2c, y2c = bT[:, 3:4], bT[:, 4:5]
        area_c = (x2c - x1c) * (y2c - y1c)

        def iou_vs_rows(br):
            x1r, y1r, x2r, y2r = br[1:2], br[2:3], br[3:4], br[4:5]
            area_r = (x2r - x1r) * (y2r - y1r)
            xx1 = jnp.maximum(x1c, x1r)
            yy1 = jnp.maximum(y1c, y1r)
            xx2 = jnp.minimum(x2c, x2r)
            yy2 = jnp.minimum(y2c, y2r)
            iw = jnp.maximum(xx2 - xx1, 0.0)
            ih = jnp.maximum(yy2 - yy1, 0.0)
            inter = iw * ih
            union = area_c + area_r - inter
            return inter / jnp.maximum(union, 1e-9)

        def jstep(j, ext):
            br = bs_ref[0 + j]
            iou = iou_vs_rows(br)
            krow = kp_ref[pl.ds(j, 1), :]
            supp = jnp.where((iou > _TH) & (krow > 0.0), 1.0, 0.0)
            return jnp.maximum(ext, jnp.max(supp, axis=1, keepdims=True))

        ext = lax.fori_loop(0, k, jstep, jnp.zeros((_LANES, 1), jnp.float32))
        avail = jnp.where((avail0 > 0.0) & (ext == 0.0), 1.0, 0.0)

        iou_cc = iou_vs_rows(rows)
        mlow = jnp.where(iou_cc > _TH, 1.0, 0.0) * lt_strict

        def fcond(c):
            return jnp.logical_not(c[1])

        def fbody(c):
            s, _ = c
            tmp = avail * (1.0 - s)
            s2 = jnp.where(
                lax.dot_general(mlow, tmp, (((1,), (0,)), ((), ())),
                                precision=_HIGH) > 0.0, 1.0, 0.0)
            return (s2, jnp.all(s2 == s))

        s0 = jnp.zeros((_LANES, 1), jnp.float32)
        s_fin, _ = lax.while_loop(fcond, fbody, (s0, jnp.asarray(False)))
        keep = avail * (1.0 - s_fin)
        return keep, bT

    def block_step(k, carry):
        @pl.when(cnt[0] < _POST)
        def _():
            avail0 = jnp.where(sub_iota + _LANES * k < valid_n, 1.0, 0.0)
            rows_l = decode_side(dl_ref, bs_l, k)
            rows_r = decode_side(dr_ref, bs_r, k)
            keep_l, bT_l = side_keep(rows_l, bs_l, kp_l, k, avail0)
            keep_r, bT_r = side_keep(rows_r, bs_r, kp_r, k, avail0)
            joint = keep_l * keep_r
            pos = lax.dot_general(lt_strict, joint, (((1,), (0,)), ((), ())),
                                  precision=_HIGH) + cnt[0].astype(jnp.float32)
            x = jnp.concatenate(
                [keep_l, keep_r, joint, pos,
                 jnp.zeros((_LANES, 4), jnp.float32)], axis=1)
            r = _tr(x)
            kp_l[pl.ds(k, 1), :] = r[0:1]
            kp_r[pl.ds(k, 1), :] = r[1:2]
            jrow = r[2:3]
            prow = r[3:4]
            onehot = jnp.where((q_iota == prow) & (jrow > 0.0), 1.0, 0.0)
            acc_l[...] += lax.dot_general(
                onehot, bT_l, (((1,), (0,)), ((), ())), precision=_HIGH)
            acc_r[...] += lax.dot_general(
                onehot, bT_r, (((1,), (0,)), ((), ())), precision=_HIGH)
            cnt[0] = cnt[0] + jnp.sum(joint).astype(jnp.int32)
        return carry

    lax.fori_loop(0, nblk, block_step, 0)

    lane5 = lax.broadcasted_iota(jnp.int32, (_POST, 8), 1)
    bi = i.astype(jnp.float32)
    final_l = jnp.where(lane5 == 0, bi, acc_l[0:_POST, :])
    final_r = jnp.where(lane5 == 0, bi, acc_r[0:_POST, :])
    out_l_ref[0] = final_l[:, 0:5]
    out_r_ref[0] = final_r[:, 0:5]
    out_cnt_ref[0] = jnp.full((8, _LANES), cnt[0], jnp.int32)


def _nms_pipeline(order, valid_n, nblk, dl, dr, imax_b):
    B = order.shape[0]
    anch = jnp.broadcast_to(jnp.asarray(_ANCHORS)[None], (B,) + _ANCHORS.shape)
    anch_g = jnp.take_along_axis(anch, order[..., None], axis=1)
    dl_g = jnp.take_along_axis(dl, order[..., None], axis=1)
    dr_g = jnp.take_along_axis(dr, order[..., None], axis=1)
    return _nms_gathered(anch_g, dl_g, dr_g, valid_n, nblk, imax_b)


def _nms_gathered(anch_g, dl_g, dr_g, valid_n, nblk, imax_b):
    B = anch_g.shape[0]

    def to_blocks(x):
        x = x.transpose(0, 2, 1).reshape(B, 4, nblk, _LANES).transpose(0, 2, 1, 3)
        return jnp.pad(x, ((0, 0), (0, 0), (1, 3), (0, 0)))

    out_l, out_r, cnts = pl.pallas_call(
        functools.partial(_nms_body, nblk, valid_n),
        grid=(B,),
        in_specs=[
            pl.BlockSpec((1, 8, _LANES), lambda i: (i, 0, 0)),
            pl.BlockSpec((1, nblk, 8, _LANES), lambda i: (i, 0, 0, 0)),
            pl.BlockSpec((1, nblk, 8, _LANES), lambda i: (i, 0, 0, 0)),
            pl.BlockSpec((1, nblk, 8, _LANES), lambda i: (i, 0, 0, 0)),
        ],
        out_specs=[
            pl.BlockSpec((1, _POST, 5), lambda i: (i, 0, 0)),
            pl.BlockSpec((1, _POST, 5), lambda i: (i, 0, 0)),
            pl.BlockSpec((1, 8, _LANES), lambda i: (i, 0, 0)),
        ],
        out_shape=[
            jax.ShapeDtypeStruct((B, _POST, 5), jnp.float32),
            jax.ShapeDtypeStruct((B, _POST, 5), jnp.float32),
            jax.ShapeDtypeStruct((B, 8, _LANES), jnp.int32),
        ],
        scratch_shapes=[
            pltpu.VMEM((nblk, 8, _LANES), jnp.float32),
            pltpu.VMEM((nblk, 8, _LANES), jnp.float32),
            pltpu.VMEM((nblk, _LANES), jnp.float32),
            pltpu.VMEM((nblk, _LANES), jnp.float32),
            pltpu.VMEM((3 * _LANES, 8), jnp.float32),
            pltpu.VMEM((3 * _LANES, 8), jnp.float32),
            pltpu.SMEM((1,), jnp.int32),
        ],
    )(imax_b, to_blocks(anch_g), to_blocks(dl_g), to_blocks(dr_g))
    return out_l, out_r, cnts


_FAST_N = 1024
_SR = 512
_NSC = 65472


def _topk_body(sc_ref, out_ref, sel_s, p_s, val_s, idx_s):
    s = sc_ref[0]
    lt512 = jnp.where(
        lax.broadcasted_iota(jnp.int32, (_SR, _SR), 0)
        > lax.broadcasted_iota(jnp.int32, (_SR, _SR), 1), 1.0, 0.0)
    ltc = jnp.where(
        lax.broadcasted_iota(jnp.int32, (_LANES, _LANES), 0)
        < lax.broadcasted_iota(jnp.int32, (_LANES, _LANES), 1), 1.0, 0.0)
    liota = lax.broadcasted_iota(jnp.int32, (1, _LANES), 1)

    def count_gt(tbits):
        tf = lax.bitcast_convert_type(
            jnp.full((8, _LANES), tbits, jnp.int32), jnp.float32)[0:1]
        return jnp.sum((s > tf).astype(jnp.int32))

    def bs_step(_, c):
        lo, hi = c
        mid = (lo + hi) // 2
        ge = count_gt(mid) >= _FAST_N
        return (jnp.where(ge, mid, lo), jnp.where(ge, hi, mid))

    lo0 = jnp.asarray(-1, jnp.int32)
    hi0 = jnp.asarray(1 << 30, jnp.int32)
    _, tbits = lax.fori_loop(0, 31, bs_step, (lo0, hi0))
    tf = lax.bitcast_convert_type(
        jnp.full((8, _LANES), tbits, jnp.int32), jnp.float32)[0:1]
    n_gt = jnp.sum((s > tf).astype(jnp.int32))
    need = (_FAST_N - n_gt).astype(jnp.float32)

    eqf = jnp.where(s == tf, 1.0, 0.0)
    hp = jax.lax.Precision.HIGHEST
    dot = functools.partial(lax.dot_general, precision=hp)
    rows_eq = dot(lt512, jnp.sum(eqf, axis=1, keepdims=True),
                  (((1,), (0,)), ((), ())))
    rank_eq = rows_eq + dot(eqf, ltc, (((1,), (0,)), ((), ())))
    sel = jnp.where(s > tf, 1.0, 0.0) + eqf * jnp.where(rank_eq < need, 1.0, 0.0)
    rows_sel = dot(lt512, jnp.sum(sel, axis=1, keepdims=True),
                   (((1,), (0,)), ((), ())))
    p = rows_sel + dot(sel, ltc, (((1,), (0,)), ((), ())))
    sel_s[...] = sel
    p_s[...] = p
    val_s[...] = jnp.zeros((16, _LANES), jnp.float32)
    idx_s[...] = jnp.zeros((16, _LANES), jnp.float32)

    def crow(r, base):
        selr = sel_s[pl.ds(r, 1), :]
        pr = p_s[pl.ds(r, 1), :]
        sr = sc_ref[0, pl.ds(r, 1), :]
        idxr = (r * _LANES + liota).astype(jnp.float32)
        x = jnp.concatenate(
            [selr, pr, jnp.zeros((6, _LANES), jnp.float32)], axis=0)
        xT = _tr(x)
        selc, pc = xT[:, 0:1], xT[:, 1:2]
        q0 = base // _LANES
        q0f = q0.astype(jnp.float32)
        vals = jnp.concatenate([sr, idxr], axis=0)
        for dq in (0, 1):
            oh = jnp.where(
                (pc - (q0f + dq) * _LANES == liota.astype(jnp.float32))
                & (selc > 0.0), 1.0, 0.0)
            res = dot(vals, oh, (((1,), (0,)), ((), ())))
            qd = q0 + dq
            val_s[pl.ds(qd, 1), :] += res[0:1]
            idx_s[pl.ds(qd, 1), :] += res[1:2]
        return base + jnp.sum(selr).astype(jnp.int32)

    lax.fori_loop(0, _SR, crow, jnp.asarray(0, jnp.int32))

    kv = val_s[0:8, :]
    iv = idx_s[0:8, :]
    riota = lax.broadcasted_iota(jnp.int32, (8, 1), 0)

    def rowroll(x, m):
        return jnp.concatenate([x[m:, :], x[:m, :]], axis=0)

    def laneroll(x, d):
        return jnp.concatenate([x[:, d:], x[:, :d]], axis=1)

    for st in range(1, 11):
        for d in [1 << b for b in range(st - 1, -1, -1)]:
            if d >= _LANES:
                m = d >> 7
                is_lo = (riota & m) == 0
                kp = jnp.where(is_lo, rowroll(kv, m), rowroll(kv, 8 - m))
                ip = jnp.where(is_lo, rowroll(iv, m), rowroll(iv, 8 - m))
            else:
                is_lo = (liota & d) == 0
                kp = jnp.where(is_lo, laneroll(kv, d), laneroll(kv, _LANES - d))
                ip = jnp.where(is_lo, laneroll(iv, d), laneroll(iv, _LANES - d))
            sbit = 1 << st
            if sbit >= _LANES:
                dirup = (riota & (sbit >> 7)) == 0
            else:
                dirup = (liota & sbit) == 0
            before = (kv > kp) | ((kv == kp) & (iv < ip))
            keep = before == (is_lo == dirup)
            kv = jnp.where(keep, kv, kp)
            iv = jnp.where(keep, iv, ip)

    out_ref[0] = iv.astype(jnp.int32)


def _topk_sorted(sf):
    B = sf.shape[0]
    sp = jnp.pad(sf, ((0, 0), (0, _SR * _LANES - _NSC)),
                 constant_values=-1.0).reshape(B, _SR, _LANES)
    out = pl.pallas_call(
        _topk_body,
        grid=(B,),
        in_specs=[pl.BlockSpec((1, _SR, _LANES), lambda i: (i, 0, 0))],
        out_specs=pl.BlockSpec((1, 8, _LANES), lambda i: (i, 0, 0)),
        out_shape=jax.ShapeDtypeStruct((B, 8, _LANES), jnp.int32),
        scratch_shapes=[
            pltpu.VMEM((_SR, _LANES), jnp.float32),
            pltpu.VMEM((_SR, _LANES), jnp.float32),
            pltpu.VMEM((16, _LANES), jnp.float32),
            pltpu.VMEM((16, _LANES), jnp.float32),
        ],
    )(sp)
    return out.reshape(B, 8 * _LANES)


def _sc_gather(deltas_flat, anchors, idx_flat, idx_anch):
    n = idx_flat.shape[0]
    nw = 32
    bpw = n // nw
    mesh = plsc.VectorSubcoreMesh(core_axis_name="c", subcore_axis_name="s")

    @functools.partial(
        pl.kernel, mesh=mesh,
        out_type=[
            jax.ShapeDtypeStruct((n, 6), jnp.float32),
            jax.ShapeDtypeStruct((n, 4), jnp.float32),
        ],
        scratch_types=[
            pltpu.VMEM((bpw,), jnp.int32),
            pltpu.VMEM((bpw,), jnp.int32),
            pltpu.VMEM((bpw, 6), jnp.float32),
            pltpu.VMEM((bpw, 4), jnp.float32),
            pltpu.SemaphoreType.DMA,
            pltpu.SemaphoreType.DMA,
        ])
    def k(d_hbm, a_hbm, if_hbm, ia_hbm, out_d, out_a,
          if_v, ia_v, drows_v, arows_v, semd, sema):
        wid = lax.axis_index("s") * 2 + lax.axis_index("c")
        base = wid * bpw
        pltpu.sync_copy(if_hbm.at[pl.ds(base, bpw)], if_v)
        pltpu.sync_copy(ia_hbm.at[pl.ds(base, bpw)], ia_v)
        cd = pltpu.async_copy(d_hbm.at[if_v], drows_v, semd)
        ca = pltpu.async_copy(a_hbm.at[ia_v], arows_v, sema)
        cd.wait()
        ca.wait()
        pltpu.sync_copy(drows_v, out_d.at[pl.ds(base, bpw)])
        pltpu.sync_copy(arows_v, out_a.at[pl.ds(base, bpw)])

    return k(deltas_flat, anchors, idx_flat, idx_anch)


def kernel(scores, bbox_deltas, im_info):
    B = scores.shape[0]
    sf = scores[:, :, 1]
    dl = bbox_deltas[..., :4]
    dr = jnp.stack([bbox_deltas[..., 4], bbox_deltas[..., 1],
                    bbox_deltas[..., 5], bbox_deltas[..., 3]], axis=-1)
    imax = jnp.stack([im_info[:, 1] - 1.0, im_info[:, 0] - 1.0], axis=1)
    imax_b = jnp.broadcast_to(
        jnp.pad(imax, ((0, 0), (0, 6)))[:, :, None], (B, 8, _LANES))

    ord_fast = _topk_sorted(sf)
    fl, fr, fc = _nms_pipeline(ord_fast, _FAST_N, _FAST_N // _LANES,
                               dl, dr, imax_b)

    def full_path():
        order = jnp.argsort(-sf, axis=1)[:, :_PRE]
        order = jnp.pad(order, ((0, 0), (0, _PAD_N - _PRE)))
        ol, og, _ = _nms_pipeline(order, _PRE, _NB, dl, dr, imax_b)
        return ol, og

    return lax.cond(jnp.all(fc[:, 0, 0] >= _POST), lambda: (fl, fr), full_path)


# speedup vs baseline: 3.9568x; 3.9568x over previous
import functools

import numpy as np
import jax
import jax.numpy as jnp
from jax import lax
from jax.experimental import pallas as pl
from jax.experimental.pallas import tpu as pltpu
from jax.experimental.pallas import tpu_sc as plsc

_FPN_ANCHOR_SCALES = [32, 64, 128, 256, 512]
_FPN_FEAT_STRIDES = [4, 8, 16, 32, 64]
_ANCHOR_RATIOS = [0.5, 1.0, 2.0]
_IM_SIZE = 512
_PRE = 6000
_POST = 300
_TH = 0.7
_LANES = 128
_NB = 48
_PAD_N = _NB * _LANES
_HIGH = lax.Precision.HIGHEST


def _gen_anchors() -> np.ndarray:
    all_boxes = []
    ratios = np.array(_ANCHOR_RATIOS, dtype=np.float64)
    for scale, stride in zip(_FPN_ANCHOR_SCALES, _FPN_FEAT_STRIDES):
        fh = _IM_SIZE // stride
        fw = _IM_SIZE // stride
        heights = scale / np.sqrt(ratios)
        widths = scale * np.sqrt(ratios)
        shifts_y = np.arange(0, fh) * stride
        shifts_x = np.arange(0, fw) * stride
        sx, sy = np.meshgrid(shifts_x, shifts_y)
        box_w, box_cx = np.meshgrid(widths, sx.flatten())
        box_h, box_cy = np.meshgrid(heights, sy.flatten())
        boxes = np.stack([box_cx - 0.5 * box_w, box_cy - 0.5 * box_h,
                          box_cx + 0.5 * box_w, box_cy + 0.5 * box_h],
                         axis=2).reshape(-1, 4)
        all_boxes.append(boxes)
    return np.concatenate(all_boxes, axis=0).astype(np.float32)


_ANCHORS = _gen_anchors()


def _tr(x):
    eye = jnp.eye(x.shape[0], dtype=jnp.float32)
    return lax.dot_general(x, eye, (((0,), (0,)), ((), ())), precision=_HIGH)


def _nms_body(nblk, valid_n, im_ref, anch_ref, dl_ref, dr_ref,
              out_l_ref, out_r_ref, out_cnt_ref,
              bs_l, bs_r, kp_l, kp_r, acc_l, acc_r, cnt):
    i = pl.program_id(0)
    cnt[0] = 0
    acc_l[...] = jnp.zeros(acc_l.shape, jnp.float32)
    acc_r[...] = jnp.zeros(acc_r.shape, jnp.float32)
    imx = im_ref[0, 0:1, :]
    imy = im_ref[0, 1:2, :]

    iota_r = lax.broadcasted_iota(jnp.int32, (_LANES, _LANES), 0)
    iota_c = lax.broadcasted_iota(jnp.int32, (_LANES, _LANES), 1)
    lt_strict = jnp.where(iota_r > iota_c, 1.0, 0.0).astype(jnp.float32)
    sub_iota = lax.broadcasted_iota(jnp.int32, (_LANES, 1), 0)
    q_iota = lax.broadcasted_iota(
        jnp.int32, (3 * _LANES, _LANES), 0).astype(jnp.float32)

    def decode_side(d_ref, bs_ref, k):
        a = anch_ref[0, k]
        d = d_ref[0, k]
        x1a, y1a, x2a, y2a = a[1:2], a[2:3], a[3:4], a[4:5]
        dx, dy, dw, dh = d[1:2], d[2:3], d[3:4], d[4:5]
        w = x2a - x1a + 1.0
        h = y2a - y1a + 1.0
        cx = x1a + 0.5 * w
        cy = y1a + 0.5 * h
        pcx = dx * w + cx
        pcy = dy * h + cy
        pw = jnp.exp(dw) * w
        ph = jnp.exp(dh) * h
        px1 = jnp.clip(pcx - 0.5 * pw, 0.0, imx)
        py1 = jnp.clip(pcy - 0.5 * ph, 0.0, imy)
        px2 = jnp.clip(pcx + 0.5 * pw, 0.0, imx)
        py2 = jnp.clip(pcy + 0.5 * ph, 0.0, imy)
        rows = jnp.concatenate(
            [jnp.zeros((1, _LANES), jnp.float32), px1, py1, px2, py2,
             jnp.zeros((3, _LANES), jnp.float32)], axis=0)
        bs_ref[pl.ds(k, 1)] = rows.reshape(1, 8, _LANES)
        return rows

    def side_keep(rows, bs_ref, kp_ref, k, avail0):
        bT = _tr(rows)
        x1c, y1c = bT[:, 1:2], bT[:, 2:3]
        x2c, y2c = bT[:, 3:4], bT[:, 4:5]
        area_c = (x2c - x1c) * (y2c - y1c)

        def iou_vs_rows(br):
            x1r, y1r, x2r, y2r = br[1:2], br[2:3], br[3:4], br[4:5]
            area_r = (x2r - x1r) * (y2r - y1r)
            xx1 = jnp.maximum(x1c, x1r)
            yy1 = jnp.maximum(y1c, y1r)
            xx2 = jnp.minimum(x2c, x2r)
            yy2 = jnp.minimum(y2c, y2r)
            iw = jnp.maximum(xx2 - xx1, 0.0)
            ih = jnp.maximum(yy2 - yy1, 0.0)
            inter = iw * ih
            union = area_c + area_r - inter
            return inter / jnp.maximum(union, 1e-9)

        def jstep(j, ext):
            br = bs_ref[0 + j]
            iou = iou_vs_rows(br)
            krow = kp_ref[pl.ds(j, 1), :]
            supp = jnp.where((iou > _TH) & (krow > 0.0), 1.0, 0.0)
            return jnp.maximum(ext, jnp.max(supp, axis=1, keepdims=True))

        ext = lax.fori_loop(0, k, jstep, jnp.zeros((_LANES, 1), jnp.float32))
        avail = jnp.where((avail0 > 0.0) & (ext == 0.0), 1.0, 0.0)

        iou_cc = iou_vs_rows(rows)
        mlow = jnp.where(iou_cc > _TH, 1.0, 0.0) * lt_strict

        def fcond(c):
            return jnp.logical_not(c[1])

        def fbody(c):
            s, _ = c
            tmp = avail * (1.0 - s)
            s2 = jnp.where(
                lax.dot_general(mlow, tmp, (((1,), (0,)), ((), ())),
                                precision=_HIGH) > 0.0, 1.0, 0.0)
            return (s2, jnp.all(s2 == s))

        s0 = jnp.zeros((_LANES, 1), jnp.float32)
        s_fin, _ = lax.while_loop(fcond, fbody, (s0, jnp.asarray(False)))
        keep = avail * (1.0 - s_fin)
        return keep, bT

    def block_step(k, carry):
        @pl.when(cnt[0] < _POST)
        def _():
            avail0 = jnp.where(sub_iota + _LANES * k < valid_n, 1.0, 0.0)
            rows_l = decode_side(dl_ref, bs_l, k)
            rows_r = decode_side(dr_ref, bs_r, k)
            keep_l, bT_l = side_keep(rows_l, bs_l, kp_l, k, avail0)
            keep_r, bT_r = side_keep(rows_r, bs_r, kp_r, k, avail0)
            joint = keep_l * keep_r
            pos = lax.dot_general(lt_strict, joint, (((1,), (0,)), ((), ())),
                                  precision=_HIGH) + cnt[0].astype(jnp.float32)
            x = jnp.concatenate(
                [keep_l, keep_r, joint, pos,
                 jnp.zeros((_LANES, 4), jnp.float32)], axis=1)
            r = _tr(x)
            kp_l[pl.ds(k, 1), :] = r[0:1]
            kp_r[pl.ds(k, 1), :] = r[1:2]
            jrow = r[2:3]
            prow = r[3:4]
            onehot = jnp.where((q_iota == prow) & (jrow > 0.0), 1.0, 0.0)
            acc_l[...] += lax.dot_general(
                onehot, bT_l, (((1,), (0,)), ((), ())), precision=_HIGH)
            acc_r[...] += lax.dot_general(
                onehot, bT_r, (((1,), (0,)), ((), ())), precision=_HIGH)
            cnt[0] = cnt[0] + jnp.sum(joint).astype(jnp.int32)
        return carry

    lax.fori_loop(0, nblk, block_step, 0)

    lane5 = lax.broadcasted_iota(jnp.int32, (_POST, 8), 1)
    bi = i.astype(jnp.float32)
    final_l = jnp.where(lane5 == 0, bi, acc_l[0:_POST, :])
    final_r = jnp.where(lane5 == 0, bi, acc_r[0:_POST, :])
    out_l_ref[0] = final_l[:, 0:5]
    out_r_ref[0] = final_r[:, 0:5]
    out_cnt_ref[0] = jnp.full((8, _LANES), cnt[0], jnp.int32)


def _nms_pipeline(order, valid_n, nblk, dl, dr, imax_b):
    B = order.shape[0]
    anch = jnp.broadcast_to(jnp.asarray(_ANCHORS)[None], (B,) + _ANCHORS.shape)
    anch_g = jnp.take_along_axis(anch, order[..., None], axis=1)
    dl_g = jnp.take_along_axis(dl, order[..., None], axis=1)
    dr_g = jnp.take_along_axis(dr, order[..., None], axis=1)
    return _nms_gathered(anch_g, dl_g, dr_g, valid_n, nblk, imax_b)


def _nms_gathered(anch_g, dl_g, dr_g, valid_n, nblk, imax_b):
    B = anch_g.shape[0]

    def to_blocks(x):
        x = x.transpose(0, 2, 1).reshape(B, 4, nblk, _LANES).transpose(0, 2, 1, 3)
        return jnp.pad(x, ((0, 0), (0, 0), (1, 3), (0, 0)))

    out_l, out_r, cnts = pl.pallas_call(
        functools.partial(_nms_body, nblk, valid_n),
        grid=(B,),
        in_specs=[
            pl.BlockSpec((1, 8, _LANES), lambda i: (i, 0, 0)),
            pl.BlockSpec((1, nblk, 8, _LANES), lambda i: (i, 0, 0, 0)),
            pl.BlockSpec((1, nblk, 8, _LANES), lambda i: (i, 0, 0, 0)),
            pl.BlockSpec((1, nblk, 8, _LANES), lambda i: (i, 0, 0, 0)),
        ],
        out_specs=[
            pl.BlockSpec((1, _POST, 5), lambda i: (i, 0, 0)),
            pl.BlockSpec((1, _POST, 5), lambda i: (i, 0, 0)),
            pl.BlockSpec((1, 8, _LANES), lambda i: (i, 0, 0)),
        ],
        out_shape=[
            jax.ShapeDtypeStruct((B, _POST, 5), jnp.float32),
            jax.ShapeDtypeStruct((B, _POST, 5), jnp.float32),
            jax.ShapeDtypeStruct((B, 8, _LANES), jnp.int32),
        ],
        scratch_shapes=[
            pltpu.VMEM((nblk, 8, _LANES), jnp.float32),
            pltpu.VMEM((nblk, 8, _LANES), jnp.float32),
            pltpu.VMEM((nblk, _LANES), jnp.float32),
            pltpu.VMEM((nblk, _LANES), jnp.float32),
            pltpu.VMEM((3 * _LANES, 8), jnp.float32),
            pltpu.VMEM((3 * _LANES, 8), jnp.float32),
            pltpu.SMEM((1,), jnp.int32),
        ],
    )(imax_b, to_blocks(anch_g), to_blocks(dl_g), to_blocks(dr_g))
    return out_l, out_r, cnts


_FAST_N = 1024
_SR = 512
_NSC = 65472


_NEX = 16
_CR = _SR // 8


def _topk_body(sc_ref, out_ref, ok_ref):
    hp = jax.lax.Precision.HIGHEST
    dot = functools.partial(lax.dot_general, precision=hp)
    mm = lambda a, b: dot(a, b, (((1,), (0,)), ((), ())))
    x = sc_ref[0]
    liota = lax.broadcasted_iota(jnp.int32, (1, _LANES), 1)
    lanecol = lax.broadcasted_iota(
        jnp.int32, (_LANES, 1), 0).astype(jnp.float32)
    ltc = jnp.where(
        lax.broadcasted_iota(jnp.int32, (_LANES, _LANES), 0)
        < lax.broadcasted_iota(jnp.int32, (_LANES, _LANES), 1), 1.0, 0.0)
    rowbase = (lax.broadcasted_iota(jnp.int32, (_SR, 1), 0)
               * _LANES).astype(jnp.float32)

    vcols, icols = [], []
    for _ in range(_NEX):
        m = jnp.max(x, axis=1, keepdims=True)
        eqf = jnp.where(x == m, 1.0, 0.0)
        fo = (eqf > 0.0) & (mm(eqf, ltc) == 0.0)
        lane = mm(jnp.where(fo, 1.0, 0.0), lanecol)
        vcols.append(m)
        icols.append(rowbase + lane)
        x = jnp.where(fo, -1.0, x)
    vals16 = jnp.concatenate(vcols, axis=1)
    idx16 = jnp.concatenate(icols, axis=1)
    eb = jnp.max(vals16[:, _NEX - 1:_NEX])

    kv = jnp.zeros((_CR, _LANES), jnp.float32)
    iv = jnp.zeros((_CR, _LANES), jnp.float32)
    r64 = lax.broadcasted_iota(jnp.int32, (_CR, _SR), 0)
    c64 = lax.broadcasted_iota(jnp.int32, (_CR, _SR), 1)
    t16 = lax.broadcasted_iota(jnp.int32, (_NEX, _LANES), 0)
    m16 = lax.broadcasted_iota(jnp.int32, (_NEX, _LANES), 1)
    for a in range(8):
        sel_a = jnp.where(c64 == 8 * r64 + a, 1.0, 0.0)
        spread_a = jnp.where(m16 == 16 * a + t16, 1.0, 0.0)
        kv = kv + mm(mm(sel_a, vals16), spread_a)
        iv = iv + mm(mm(sel_a, idx16), spread_a)

    riota = lax.broadcasted_iota(jnp.int32, (_CR, 1), 0)

    def rowroll(z, m_):
        return jnp.concatenate([z[m_:, :], z[:m_, :]], axis=0)

    def laneroll(z, d_):
        return jnp.concatenate([z[:, d_:], z[:, :d_]], axis=1)

    for st in range(1, 14):
        for d in [1 << b for b in range(st - 1, -1, -1)]:
            if d >= _LANES:
                mr = d >> 7
                is_lo = (riota & mr) == 0
                kp = jnp.where(is_lo, rowroll(kv, mr), rowroll(kv, _CR - mr))
                ip = jnp.where(is_lo, rowroll(iv, mr), rowroll(iv, _CR - mr))
            else:
                is_lo = (liota & d) == 0
                kp = jnp.where(is_lo, laneroll(kv, d), laneroll(kv, _LANES - d))
                ip = jnp.where(is_lo, laneroll(iv, d), laneroll(iv, _LANES - d))
            sbit = 1 << st
            if sbit >= _LANES:
                dirup = (riota & (sbit >> 7)) == 0
            else:
                dirup = (liota & sbit) == 0
            before = (kv > kp) | ((kv == kp) & (iv < ip))
            keep = before == (is_lo == dirup)
            kv = jnp.where(keep, kv, kp)
            iv = jnp.where(keep, iv, ip)

    kth = kv[7:8, _LANES - 1:_LANES]
    ok = (kth > eb).astype(jnp.int32)
    out_ref[0] = iv[0:8, :].astype(jnp.int32)
    ok_ref[0] = jnp.broadcast_to(ok, (8, _LANES))


def _topk_sorted(sf):
    B = sf.shape[0]
    sp = jnp.pad(sf, ((0, 0), (0, _SR * _LANES - _NSC)),
                 constant_values=-1.0).reshape(B, _SR, _LANES)
    out, ok = pl.pallas_call(
        _topk_body,
        grid=(B,),
        in_specs=[pl.BlockSpec((1, _SR, _LANES), lambda i: (i, 0, 0))],
        out_specs=[
            pl.BlockSpec((1, 8, _LANES), lambda i: (i, 0, 0)),
            pl.BlockSpec((1, 8, _LANES), lambda i: (i, 0, 0)),
        ],
        out_shape=[
            jax.ShapeDtypeStruct((B, 8, _LANES), jnp.int32),
            jax.ShapeDtypeStruct((B, 8, _LANES), jnp.int32),
        ],
    )(sp)
    return out.reshape(B, 8 * _LANES), ok[:, 0, 0]


def _sc_gather(deltas_flat, anchors, idx_flat, idx_anch):
    n = idx_flat.shape[0]
    nw = 32
    bpw = n // nw
    mesh = plsc.VectorSubcoreMesh(core_axis_name="c", subcore_axis_name="s")

    @functools.partial(
        pl.kernel, mesh=mesh,
        compiler_params=pltpu.CompilerParams(use_tc_tiling_on_sc=False),
        out_type=[
            jax.ShapeDtypeStruct((n, 8), jnp.float32),
            jax.ShapeDtypeStruct((n, 8), jnp.float32),
        ],
        scratch_types=[
            pltpu.VMEM((bpw,), jnp.int32),
            pltpu.VMEM((bpw,), jnp.int32),
            pltpu.VMEM((bpw, 8), jnp.float32),
            pltpu.VMEM((bpw, 8), jnp.float32),
            pltpu.SemaphoreType.DMA,
            pltpu.SemaphoreType.DMA,
        ])
    def k(d_hbm, a_hbm, if_hbm, ia_hbm, out_d, out_a,
          if_v, ia_v, drows_v, arows_v, semd, sema):
        wid = lax.axis_index("s") * 2 + lax.axis_index("c")
        base = wid * bpw
        pltpu.sync_copy(if_hbm.at[pl.ds(base, bpw)], if_v)
        pltpu.sync_copy(ia_hbm.at[pl.ds(base, bpw)], ia_v)
        cd = pltpu.async_copy(d_hbm.at[if_v], drows_v, semd)
        ca = pltpu.async_copy(a_hbm.at[ia_v], arows_v, sema)
        cd.wait()
        ca.wait()
        pltpu.sync_copy(drows_v, out_d.at[pl.ds(base, bpw)])
        pltpu.sync_copy(arows_v, out_a.at[pl.ds(base, bpw)])

    return k(deltas_flat, anchors, idx_flat, idx_anch)


def kernel(scores, bbox_deltas, im_info):
    B = scores.shape[0]
    sf = scores[:, :, 1]
    dl = bbox_deltas[..., :4]
    dr = jnp.stack([bbox_deltas[..., 4], bbox_deltas[..., 1],
                    bbox_deltas[..., 5], bbox_deltas[..., 3]], axis=-1)
    imax = jnp.stack([im_info[:, 1] - 1.0, im_info[:, 0] - 1.0], axis=1)
    imax_b = jnp.broadcast_to(
        jnp.pad(imax, ((0, 0), (0, 6)))[:, :, None], (B, 8, _LANES))

    ord_fast, ok_topk = _topk_sorted(sf)
    boff = (jnp.arange(B, dtype=jnp.int32) * _NSC)[:, None]
    deltas8 = jnp.pad(bbox_deltas, ((0, 0), (0, 0), (0, 2))).reshape(
        B * _NSC, 8)
    anch8 = jnp.asarray(np.pad(_ANCHORS, ((0, 0), (0, 4))))
    d_g, a_g = _sc_gather(deltas8, anch8,
                          (ord_fast + boff).reshape(-1),
                          ord_fast.reshape(-1))
    anch_g = a_g[:, :4].reshape(B, _FAST_N, 4)
    dlf_g = d_g[:, :4].reshape(B, _FAST_N, 4)
    drf_g = jnp.stack([d_g[:, 4], d_g[:, 1], d_g[:, 5], d_g[:, 3]],
                      axis=-1).reshape(B, _FAST_N, 4)
    fl, fr, fc = _nms_gathered(anch_g, dlf_g, drf_g, _FAST_N,
                               _FAST_N // _LANES, imax_b)

    def full_path():
        order = jnp.argsort(-sf, axis=1)[:, :_PRE]
        order = jnp.pad(order, ((0, 0), (0, _PAD_N - _PRE)))
        ol, og, _ = _nms_pipeline(order, _PRE, _NB, dl, dr, imax_b)
        return ol, og

    good = jnp.all(fc[:, 0, 0] >= _POST) & jnp.all(ok_topk == 1)
    return lax.cond(good, lambda: (fl, fr), full_path)


# speedup vs baseline: 4.0037x; 1.0118x over previous
import functools

import numpy as np
import jax
import jax.numpy as jnp
from jax import lax
from jax.experimental import pallas as pl
from jax.experimental.pallas import tpu as pltpu
from jax.experimental.pallas import tpu_sc as plsc

_FPN_ANCHOR_SCALES = [32, 64, 128, 256, 512]
_FPN_FEAT_STRIDES = [4, 8, 16, 32, 64]
_ANCHOR_RATIOS = [0.5, 1.0, 2.0]
_IM_SIZE = 512
_PRE = 6000
_POST = 300
_TH = 0.7
_LANES = 128
_NB = 48
_PAD_N = _NB * _LANES
_HIGH = lax.Precision.HIGHEST


def _gen_anchors() -> np.ndarray:
    all_boxes = []
    ratios = np.array(_ANCHOR_RATIOS, dtype=np.float64)
    for scale, stride in zip(_FPN_ANCHOR_SCALES, _FPN_FEAT_STRIDES):
        fh = _IM_SIZE // stride
        fw = _IM_SIZE // stride
        heights = scale / np.sqrt(ratios)
        widths = scale * np.sqrt(ratios)
        shifts_y = np.arange(0, fh) * stride
        shifts_x = np.arange(0, fw) * stride
        sx, sy = np.meshgrid(shifts_x, shifts_y)
        box_w, box_cx = np.meshgrid(widths, sx.flatten())
        box_h, box_cy = np.meshgrid(heights, sy.flatten())
        boxes = np.stack([box_cx - 0.5 * box_w, box_cy - 0.5 * box_h,
                          box_cx + 0.5 * box_w, box_cy + 0.5 * box_h],
                         axis=2).reshape(-1, 4)
        all_boxes.append(boxes)
    return np.concatenate(all_boxes, axis=0).astype(np.float32)


_ANCHORS = _gen_anchors()


def _tr(x):
    eye = jnp.eye(x.shape[0], dtype=jnp.float32)
    return lax.dot_general(x, eye, (((0,), (0,)), ((), ())), precision=_HIGH)


def _nms_body(nblk, valid_n, im_ref, anch_ref, dl_ref, dr_ref,
              out_l_ref, out_r_ref, out_cnt_ref,
              bs_l, bs_r, kp_l, kp_r, acc_l, acc_r, cnt):
    i = pl.program_id(0)
    cnt[0] = 0
    acc_l[...] = jnp.zeros(acc_l.shape, jnp.float32)
    acc_r[...] = jnp.zeros(acc_r.shape, jnp.float32)
    imx = im_ref[0, 0:1, :]
    imy = im_ref[0, 1:2, :]

    iota_r = lax.broadcasted_iota(jnp.int32, (_LANES, _LANES), 0)
    iota_c = lax.broadcasted_iota(jnp.int32, (_LANES, _LANES), 1)
    lt_strict = jnp.where(iota_r > iota_c, 1.0, 0.0).astype(jnp.float32)
    sub_iota = lax.broadcasted_iota(jnp.int32, (_LANES, 1), 0)
    q_iota = lax.broadcasted_iota(
        jnp.int32, (3 * _LANES, _LANES), 0).astype(jnp.float32)

    def decode_side(d_ref, bs_ref, k):
        a = anch_ref[0, k]
        d = d_ref[0, k]
        x1a, y1a, x2a, y2a = a[1:2], a[2:3], a[3:4], a[4:5]
        dx, dy, dw, dh = d[1:2], d[2:3], d[3:4], d[4:5]
        w = x2a - x1a + 1.0
        h = y2a - y1a + 1.0
        cx = x1a + 0.5 * w
        cy = y1a + 0.5 * h
        pcx = dx * w + cx
        pcy = dy * h + cy
        pw = jnp.exp(dw) * w
        ph = jnp.exp(dh) * h
        px1 = jnp.clip(pcx - 0.5 * pw, 0.0, imx)
        py1 = jnp.clip(pcy - 0.5 * ph, 0.0, imy)
        px2 = jnp.clip(pcx + 0.5 * pw, 0.0, imx)
        py2 = jnp.clip(pcy + 0.5 * ph, 0.0, imy)
        rows = jnp.concatenate(
            [jnp.zeros((1, _LANES), jnp.float32), px1, py1, px2, py2,
             jnp.zeros((3, _LANES), jnp.float32)], axis=0)
        bs_ref[pl.ds(k, 1)] = rows.reshape(1, 8, _LANES)
        return rows

    def side_keep(rows, bs_ref, kp_ref, k, avail0):
        bT = _tr(rows)
        x1c, y1c = bT[:, 1:2], bT[:, 2:3]
        x2c, y2c = bT[:, 3:4], bT[:, 4:5]
        area_c = (x2c - x1c) * (y2c - y1c)

        def iou_vs_rows(br):
            x1r, y1r, x2r, y2r = br[1:2], br[2:3], br[3:4], br[4:5]
            area_r = (x2r - x1r) * (y2r - y1r)
            xx1 = jnp.maximum(x1c, x1r)
            yy1 = jnp.maximum(y1c, y1r)
            xx2 = jnp.minimum(x2c, x2r)
            yy2 = jnp.minimum(y2c, y2r)
            iw = jnp.maximum(xx2 - xx1, 0.0)
            ih = jnp.maximum(yy2 - yy1, 0.0)
            inter = iw * ih
            union = area_c + area_r - inter
            return inter / jnp.maximum(union, 1e-9)

        def jstep(j, ext):
            br = bs_ref[0 + j]
            iou = iou_vs_rows(br)
            krow = kp_ref[pl.ds(j, 1), :]
            supp = jnp.where((iou > _TH) & (krow > 0.0), 1.0, 0.0)
            return jnp.maximum(ext, jnp.max(supp, axis=1, keepdims=True))

        ext = lax.fori_loop(0, k, jstep, jnp.zeros((_LANES, 1), jnp.float32))
        avail = jnp.where((avail0 > 0.0) & (ext == 0.0), 1.0, 0.0)

        iou_cc = iou_vs_rows(rows)
        mlow = jnp.where(iou_cc > _TH, 1.0, 0.0) * lt_strict

        def fcond(c):
            return jnp.logical_not(c[1])

        def fbody(c):
            s, _ = c
            tmp = avail * (1.0 - s)
            s2 = jnp.where(
                lax.dot_general(mlow, tmp, (((1,), (0,)), ((), ())),
                                precision=_HIGH) > 0.0, 1.0, 0.0)
            return (s2, jnp.all(s2 == s))

        s0 = jnp.zeros((_LANES, 1), jnp.float32)
        s_fin, _ = lax.while_loop(fcond, fbody, (s0, jnp.asarray(False)))
        keep = avail * (1.0 - s_fin)
        return keep, bT

    def block_step(k, carry):
        @pl.when(cnt[0] < _POST)
        def _():
            avail0 = jnp.where(sub_iota + _LANES * k < valid_n, 1.0, 0.0)
            rows_l = decode_side(dl_ref, bs_l, k)
            rows_r = decode_side(dr_ref, bs_r, k)
            keep_l, bT_l = side_keep(rows_l, bs_l, kp_l, k, avail0)
            keep_r, bT_r = side_keep(rows_r, bs_r, kp_r, k, avail0)
            joint = keep_l * keep_r
            pos = lax.dot_general(lt_strict, joint, (((1,), (0,)), ((), ())),
                                  precision=_HIGH) + cnt[0].astype(jnp.float32)
            x = jnp.concatenate(
                [keep_l, keep_r, joint, pos,
                 jnp.zeros((_LANES, 4), jnp.float32)], axis=1)
            r = _tr(x)
            kp_l[pl.ds(k, 1), :] = r[0:1]
            kp_r[pl.ds(k, 1), :] = r[1:2]
            jrow = r[2:3]
            prow = r[3:4]
            onehot = jnp.where((q_iota == prow) & (jrow > 0.0), 1.0, 0.0)
            acc_l[...] += lax.dot_general(
                onehot, bT_l, (((1,), (0,)), ((), ())), precision=_HIGH)
            acc_r[...] += lax.dot_general(
                onehot, bT_r, (((1,), (0,)), ((), ())), precision=_HIGH)
            cnt[0] = cnt[0] + jnp.sum(joint).astype(jnp.int32)
        return carry

    lax.fori_loop(0, nblk, block_step, 0)

    lane5 = lax.broadcasted_iota(jnp.int32, (_POST, 8), 1)
    bi = i.astype(jnp.float32)
    final_l = jnp.where(lane5 == 0, bi, acc_l[0:_POST, :])
    final_r = jnp.where(lane5 == 0, bi, acc_r[0:_POST, :])
    out_l_ref[0] = final_l[:, 0:5]
    out_r_ref[0] = final_r[:, 0:5]
    out_cnt_ref[0] = jnp.full((8, _LANES), cnt[0], jnp.int32)


def _nms_pipeline(order, valid_n, nblk, dl, dr, imax_b):
    B = order.shape[0]
    anch = jnp.broadcast_to(jnp.asarray(_ANCHORS)[None], (B,) + _ANCHORS.shape)
    anch_g = jnp.take_along_axis(anch, order[..., None], axis=1)
    dl_g = jnp.take_along_axis(dl, order[..., None], axis=1)
    dr_g = jnp.take_along_axis(dr, order[..., None], axis=1)
    return _nms_gathered(anch_g, dl_g, dr_g, valid_n, nblk, imax_b)


def _nms_gathered(anch_g, dl_g, dr_g, valid_n, nblk, imax_b):
    B = anch_g.shape[0]

    def to_blocks(x):
        x = x.transpose(0, 2, 1).reshape(B, 4, nblk, _LANES).transpose(0, 2, 1, 3)
        return jnp.pad(x, ((0, 0), (0, 0), (1, 3), (0, 0)))

    out_l, out_r, cnts = pl.pallas_call(
        functools.partial(_nms_body, nblk, valid_n),
        grid=(B,),
        in_specs=[
            pl.BlockSpec((1, 8, _LANES), lambda i: (i, 0, 0)),
            pl.BlockSpec((1, nblk, 8, _LANES), lambda i: (i, 0, 0, 0)),
            pl.BlockSpec((1, nblk, 8, _LANES), lambda i: (i, 0, 0, 0)),
            pl.BlockSpec((1, nblk, 8, _LANES), lambda i: (i, 0, 0, 0)),
        ],
        out_specs=[
            pl.BlockSpec((1, _POST, 5), lambda i: (i, 0, 0)),
            pl.BlockSpec((1, _POST, 5), lambda i: (i, 0, 0)),
            pl.BlockSpec((1, 8, _LANES), lambda i: (i, 0, 0)),
        ],
        out_shape=[
            jax.ShapeDtypeStruct((B, _POST, 5), jnp.float32),
            jax.ShapeDtypeStruct((B, _POST, 5), jnp.float32),
            jax.ShapeDtypeStruct((B, 8, _LANES), jnp.int32),
        ],
        scratch_shapes=[
            pltpu.VMEM((nblk, 8, _LANES), jnp.float32),
            pltpu.VMEM((nblk, 8, _LANES), jnp.float32),
            pltpu.VMEM((nblk, _LANES), jnp.float32),
            pltpu.VMEM((nblk, _LANES), jnp.float32),
            pltpu.VMEM((3 * _LANES, 8), jnp.float32),
            pltpu.VMEM((3 * _LANES, 8), jnp.float32),
            pltpu.SMEM((1,), jnp.int32),
        ],
    )(imax_b, to_blocks(anch_g), to_blocks(dl_g), to_blocks(dr_g))
    return out_l, out_r, cnts


_FAST_N = 1024
_SR = 512
_NSC = 65472


_NEX = 16
_CR = _SR // 8


def _topk_body(sc_ref, out_ref, ok_ref):
    hp = jax.lax.Precision.HIGHEST
    dot = functools.partial(lax.dot_general, precision=hp)
    mm = lambda a, b: dot(a, b, (((1,), (0,)), ((), ())))
    x = sc_ref[0]
    liota = lax.broadcasted_iota(jnp.int32, (1, _LANES), 1)
    lanecol = lax.broadcasted_iota(
        jnp.int32, (_LANES, 1), 0).astype(jnp.float32)
    ltc = jnp.where(
        lax.broadcasted_iota(jnp.int32, (_LANES, _LANES), 0)
        < lax.broadcasted_iota(jnp.int32, (_LANES, _LANES), 1), 1.0, 0.0)
    rowbase = (lax.broadcasted_iota(jnp.int32, (_SR, 1), 0)
               * _LANES).astype(jnp.float32)

    vcols, icols = [], []
    for _ in range(_NEX):
        m = jnp.max(x, axis=1, keepdims=True)
        eqf = jnp.where(x == m, 1.0, 0.0)
        fo = (eqf > 0.0) & (mm(eqf, ltc) == 0.0)
        lane = mm(jnp.where(fo, 1.0, 0.0), lanecol)
        vcols.append(m)
        icols.append(rowbase + lane)
        x = jnp.where(fo, -1.0, x)
    vals16 = jnp.concatenate(vcols, axis=1)
    idx16 = jnp.concatenate(icols, axis=1)
    eb = jnp.max(vals16[:, _NEX - 1:_NEX])

    kv = jnp.zeros((_CR, _LANES), jnp.float32)
    iv = jnp.zeros((_CR, _LANES), jnp.float32)
    r64 = lax.broadcasted_iota(jnp.int32, (_CR, _SR), 0)
    c64 = lax.broadcasted_iota(jnp.int32, (_CR, _SR), 1)
    t16 = lax.broadcasted_iota(jnp.int32, (_NEX, _LANES), 0)
    m16 = lax.broadcasted_iota(jnp.int32, (_NEX, _LANES), 1)
    for a in range(8):
        sel_a = jnp.where(c64 == 8 * r64 + a, 1.0, 0.0)
        spread_a = jnp.where(m16 == 16 * a + t16, 1.0, 0.0)
        kv = kv + mm(mm(sel_a, vals16), spread_a)
        iv = iv + mm(mm(sel_a, idx16), spread_a)

    riota = lax.broadcasted_iota(jnp.int32, (_CR, 1), 0)

    def rowroll(z, m_):
        return jnp.concatenate([z[m_:, :], z[:m_, :]], axis=0)

    def laneroll(z, d_):
        return jnp.concatenate([z[:, d_:], z[:, :d_]], axis=1)

    for st in range(1, 14):
        for d in [1 << b for b in range(st - 1, -1, -1)]:
            if d >= _LANES:
                mr = d >> 7
                is_lo = (riota & mr) == 0
                kp = jnp.where(is_lo, rowroll(kv, mr), rowroll(kv, _CR - mr))
                ip = jnp.where(is_lo, rowroll(iv, mr), rowroll(iv, _CR - mr))
            else:
                is_lo = (liota & d) == 0
                kp = jnp.where(is_lo, laneroll(kv, d), laneroll(kv, _LANES - d))
                ip = jnp.where(is_lo, laneroll(iv, d), laneroll(iv, _LANES - d))
            sbit = 1 << st
            if sbit >= _LANES:
                dirup = (riota & (sbit >> 7)) == 0
            else:
                dirup = (liota & sbit) == 0
            before = (kv > kp) | ((kv == kp) & (iv < ip))
            keep = before == (is_lo == dirup)
            kv = jnp.where(keep, kv, kp)
            iv = jnp.where(keep, iv, ip)

    kth = kv[7:8, _LANES - 1:_LANES]
    ok = (kth > eb).astype(jnp.int32)
    out_ref[0] = iv[0:8, :].astype(jnp.int32)
    ok_ref[0] = jnp.broadcast_to(ok, (8, _LANES))


def _topk_sorted(sf):
    B = sf.shape[0]
    sp = jnp.pad(sf, ((0, 0), (0, _SR * _LANES - _NSC)),
                 constant_values=-1.0).reshape(B, _SR, _LANES)
    out, ok = pl.pallas_call(
        _topk_body,
        grid=(B,),
        in_specs=[pl.BlockSpec((1, _SR, _LANES), lambda i: (i, 0, 0))],
        out_specs=[
            pl.BlockSpec((1, 8, _LANES), lambda i: (i, 0, 0)),
            pl.BlockSpec((1, 8, _LANES), lambda i: (i, 0, 0)),
        ],
        out_shape=[
            jax.ShapeDtypeStruct((B, 8, _LANES), jnp.int32),
            jax.ShapeDtypeStruct((B, 8, _LANES), jnp.int32),
        ],
    )(sp)
    return out.reshape(B, 8 * _LANES), ok[:, 0, 0]


def _sc_gather(deltas_flat, anchors, idx_flat, idx_anch):
    n = idx_flat.shape[0]
    nw = 32
    bpw = n // nw
    mesh = plsc.VectorSubcoreMesh(core_axis_name="c", subcore_axis_name="s")

    @functools.partial(
        pl.kernel, mesh=mesh,
        compiler_params=pltpu.CompilerParams(use_tc_tiling_on_sc=False),
        out_type=[
            jax.ShapeDtypeStruct((n, 8), jnp.float32),
            jax.ShapeDtypeStruct((n, 8), jnp.float32),
        ],
        scratch_types=[
            pltpu.VMEM((bpw,), jnp.int32),
            pltpu.VMEM((bpw,), jnp.int32),
            pltpu.VMEM((bpw, 8), jnp.float32),
            pltpu.VMEM((bpw, 8), jnp.float32),
            pltpu.SemaphoreType.DMA,
            pltpu.SemaphoreType.DMA,
        ])
    def k(d_hbm, a_hbm, if_hbm, ia_hbm, out_d, out_a,
          if_v, ia_v, drows_v, arows_v, semd, sema):
        wid = lax.axis_index("s") * 2 + lax.axis_index("c")
        base = wid * bpw
        pltpu.sync_copy(if_hbm.at[pl.ds(base, bpw)], if_v)
        pltpu.sync_copy(ia_hbm.at[pl.ds(base, bpw)], ia_v)
        cd = pltpu.async_copy(d_hbm.at[if_v], drows_v, semd)
        ca = pltpu.async_copy(a_hbm.at[ia_v], arows_v, sema)
        cd.wait()
        ca.wait()
        pltpu.sync_copy(drows_v, out_d.at[pl.ds(base, bpw)])
        pltpu.sync_copy(arows_v, out_a.at[pl.ds(base, bpw)])

    return k(deltas_flat, anchors, idx_flat, idx_anch)


def kernel(scores, bbox_deltas, im_info):
    B = scores.shape[0]
    sf = scores[:, :, 1]
    imax = jnp.stack([im_info[:, 1] - 1.0, im_info[:, 0] - 1.0], axis=1)
    imax_b = jnp.broadcast_to(
        jnp.pad(imax, ((0, 0), (0, 6)))[:, :, None], (B, 8, _LANES))

    ord_fast, ok_topk = _topk_sorted(sf)
    boff = (jnp.arange(B, dtype=jnp.int32) * _NSC)[:, None]
    deltas8 = jnp.pad(bbox_deltas, ((0, 0), (0, 0), (0, 2))).reshape(
        B * _NSC, 8)
    anch8 = jnp.asarray(np.pad(_ANCHORS, ((0, 0), (0, 4))))
    d_g, a_g = _sc_gather(deltas8, anch8,
                          (ord_fast + boff).reshape(-1),
                          ord_fast.reshape(-1))
    anch_g = a_g[:, :4].reshape(B, _FAST_N, 4)
    dlf_g = d_g[:, :4].reshape(B, _FAST_N, 4)
    drf_g = jnp.stack([d_g[:, 4], d_g[:, 1], d_g[:, 5], d_g[:, 3]],
                      axis=-1).reshape(B, _FAST_N, 4)
    fl, fr, fc = _nms_gathered(anch_g, dlf_g, drf_g, _FAST_N,
                               _FAST_N // _LANES, imax_b)

    def full_path():
        dl = bbox_deltas[..., :4]
        dr = jnp.stack([bbox_deltas[..., 4], bbox_deltas[..., 1],
                        bbox_deltas[..., 5], bbox_deltas[..., 3]], axis=-1)
        order = jnp.argsort(-sf, axis=1)[:, :_PRE]
        order = jnp.pad(order, ((0, 0), (0, _PAD_N - _PRE)))
        ol, og, _ = _nms_pipeline(order, _PRE, _NB, dl, dr, imax_b)
        return ol, og

    good = jnp.all(fc[:, 0, 0] >= _POST) & jnp.all(ok_topk == 1)
    return lax.cond(good, lambda: (fl, fr), full_path)


# speedup vs baseline: 4.2159x; 1.0530x over previous
import functools

import numpy as np
import jax
import jax.numpy as jnp
from jax import lax
from jax.experimental import pallas as pl
from jax.experimental.pallas import tpu as pltpu
from jax.experimental.pallas import tpu_sc as plsc

_FPN_ANCHOR_SCALES = [32, 64, 128, 256, 512]
_FPN_FEAT_STRIDES = [4, 8, 16, 32, 64]
_ANCHOR_RATIOS = [0.5, 1.0, 2.0]
_IM_SIZE = 512
_PRE = 6000
_POST = 300
_TH = 0.7
_LANES = 128
_NB = 48
_PAD_N = _NB * _LANES
_HIGH = lax.Precision.HIGHEST


def _gen_anchors() -> np.ndarray:
    all_boxes = []
    ratios = np.array(_ANCHOR_RATIOS, dtype=np.float64)
    for scale, stride in zip(_FPN_ANCHOR_SCALES, _FPN_FEAT_STRIDES):
        fh = _IM_SIZE // stride
        fw = _IM_SIZE // stride
        heights = scale / np.sqrt(ratios)
        widths = scale * np.sqrt(ratios)
        shifts_y = np.arange(0, fh) * stride
        shifts_x = np.arange(0, fw) * stride
        sx, sy = np.meshgrid(shifts_x, shifts_y)
        box_w, box_cx = np.meshgrid(widths, sx.flatten())
        box_h, box_cy = np.meshgrid(heights, sy.flatten())
        boxes = np.stack([box_cx - 0.5 * box_w, box_cy - 0.5 * box_h,
                          box_cx + 0.5 * box_w, box_cy + 0.5 * box_h],
                         axis=2).reshape(-1, 4)
        all_boxes.append(boxes)
    return np.concatenate(all_boxes, axis=0).astype(np.float32)


_ANCHORS = _gen_anchors()


def _tr(x):
    eye = jnp.eye(x.shape[0], dtype=jnp.float32)
    return lax.dot_general(x, eye, (((0,), (0,)), ((), ())), precision=_HIGH)


def _nms_body(nblk, valid_n, im_ref, anch_ref, dl_ref, dr_ref,
              out_l_ref, out_r_ref, out_cnt_ref,
              bs_l, bs_r, kp_l, kp_r, acc_l, acc_r, cnt):
    i = pl.program_id(0)
    cnt[0] = 0
    acc_l[...] = jnp.zeros(acc_l.shape, jnp.float32)
    acc_r[...] = jnp.zeros(acc_r.shape, jnp.float32)
    imx = im_ref[0, 0:1, :]
    imy = im_ref[0, 1:2, :]

    iota_r = lax.broadcasted_iota(jnp.int32, (_LANES, _LANES), 0)
    iota_c = lax.broadcasted_iota(jnp.int32, (_LANES, _LANES), 1)
    lt_strict = jnp.where(iota_r > iota_c, 1.0, 0.0).astype(jnp.float32)
    sub_iota = lax.broadcasted_iota(jnp.int32, (_LANES, 1), 0)
    q_iota = lax.broadcasted_iota(
        jnp.int32, (3 * _LANES, _LANES), 0).astype(jnp.float32)

    def decode_side(d_ref, bs_ref, k):
        a = anch_ref[0, k]
        d = d_ref[0, k]
        x1a, y1a, x2a, y2a = a[1:2], a[2:3], a[3:4], a[4:5]
        dx, dy, dw, dh = d[1:2], d[2:3], d[3:4], d[4:5]
        w = x2a - x1a + 1.0
        h = y2a - y1a + 1.0
        cx = x1a + 0.5 * w
        cy = y1a + 0.5 * h
        pcx = dx * w + cx
        pcy = dy * h + cy
        pw = jnp.exp(dw) * w
        ph = jnp.exp(dh) * h
        px1 = jnp.clip(pcx - 0.5 * pw, 0.0, imx)
        py1 = jnp.clip(pcy - 0.5 * ph, 0.0, imy)
        px2 = jnp.clip(pcx + 0.5 * pw, 0.0, imx)
        py2 = jnp.clip(pcy + 0.5 * ph, 0.0, imy)
        rows = jnp.concatenate(
            [jnp.zeros((1, _LANES), jnp.float32), px1, py1, px2, py2,
             jnp.zeros((3, _LANES), jnp.float32)], axis=0)
        bs_ref[pl.ds(k, 1)] = rows.reshape(1, 8, _LANES)
        return rows

    def side_keep(rows, bs_ref, kp_ref, k, avail0):
        bT = _tr(rows)
        x1c, y1c = bT[:, 1:2], bT[:, 2:3]
        x2c, y2c = bT[:, 3:4], bT[:, 4:5]
        area_c = (x2c - x1c) * (y2c - y1c)

        def iou_vs_rows(br):
            x1r, y1r, x2r, y2r = br[1:2], br[2:3], br[3:4], br[4:5]
            area_r = (x2r - x1r) * (y2r - y1r)
            xx1 = jnp.maximum(x1c, x1r)
            yy1 = jnp.maximum(y1c, y1r)
            xx2 = jnp.minimum(x2c, x2r)
            yy2 = jnp.minimum(y2c, y2r)
            iw = jnp.maximum(xx2 - xx1, 0.0)
            ih = jnp.maximum(yy2 - yy1, 0.0)
            inter = iw * ih
            union = area_c + area_r - inter
            return inter / jnp.maximum(union, 1e-9)

        def jstep(j, ext):
            br = bs_ref[0 + j]
            iou = iou_vs_rows(br)
            krow = kp_ref[pl.ds(j, 1), :]
            supp = jnp.where((iou > _TH) & (krow > 0.0), 1.0, 0.0)
            return jnp.maximum(ext, jnp.max(supp, axis=1, keepdims=True))

        ext = lax.fori_loop(0, k, jstep, jnp.zeros((_LANES, 1), jnp.float32))
        avail = jnp.where((avail0 > 0.0) & (ext == 0.0), 1.0, 0.0)

        iou_cc = iou_vs_rows(rows)
        mlow = jnp.where(iou_cc > _TH, 1.0, 0.0) * lt_strict

        def fcond(c):
            return jnp.logical_not(c[1])

        def fbody(c):
            s, _ = c
            tmp = avail * (1.0 - s)
            s2 = jnp.where(
                lax.dot_general(mlow, tmp, (((1,), (0,)), ((), ()))) > 0.0,
                1.0, 0.0)
            return (s2, jnp.all(s2 == s))

        s0 = jnp.zeros((_LANES, 1), jnp.float32)
        s_fin, _ = lax.while_loop(fcond, fbody, (s0, jnp.asarray(False)))
        keep = avail * (1.0 - s_fin)
        return keep, bT

    def block_step(k, carry):
        @pl.when(cnt[0] < _POST)
        def _():
            avail0 = jnp.where(sub_iota + _LANES * k < valid_n, 1.0, 0.0)
            rows_l = decode_side(dl_ref, bs_l, k)
            rows_r = decode_side(dr_ref, bs_r, k)
            keep_l, bT_l = side_keep(rows_l, bs_l, kp_l, k, avail0)
            keep_r, bT_r = side_keep(rows_r, bs_r, kp_r, k, avail0)
            joint = keep_l * keep_r
            pos = lax.dot_general(lt_strict, joint, (((1,), (0,)), ((), ()))
                                  ) + cnt[0].astype(jnp.float32)
            x = jnp.concatenate(
                [keep_l, keep_r, joint, pos,
                 jnp.zeros((_LANES, 4), jnp.float32)], axis=1)
            r = _tr(x)
            kp_l[pl.ds(k, 1), :] = r[0:1]
            kp_r[pl.ds(k, 1), :] = r[1:2]
            jrow = r[2:3]
            prow = r[3:4]
            onehot = jnp.where((q_iota == prow) & (jrow > 0.0), 1.0, 0.0)
            acc_l[...] += lax.dot_general(
                onehot, bT_l, (((1,), (0,)), ((), ())), precision=_HIGH)
            acc_r[...] += lax.dot_general(
                onehot, bT_r, (((1,), (0,)), ((), ())), precision=_HIGH)
            cnt[0] = cnt[0] + jnp.sum(joint).astype(jnp.int32)
        return carry

    lax.fori_loop(0, nblk, block_step, 0)

    lane5 = lax.broadcasted_iota(jnp.int32, (_POST, 8), 1)
    bi = i.astype(jnp.float32)
    final_l = jnp.where(lane5 == 0, bi, acc_l[0:_POST, :])
    final_r = jnp.where(lane5 == 0, bi, acc_r[0:_POST, :])
    out_l_ref[0] = final_l[:, 0:5]
    out_r_ref[0] = final_r[:, 0:5]
    out_cnt_ref[0] = jnp.full((8, _LANES), cnt[0], jnp.int32)


def _nms_pipeline(order, valid_n, nblk, dl, dr, imax_b):
    B = order.shape[0]
    anch = jnp.broadcast_to(jnp.asarray(_ANCHORS)[None], (B,) + _ANCHORS.shape)
    anch_g = jnp.take_along_axis(anch, order[..., None], axis=1)
    dl_g = jnp.take_along_axis(dl, order[..., None], axis=1)
    dr_g = jnp.take_along_axis(dr, order[..., None], axis=1)
    return _nms_gathered(anch_g, dl_g, dr_g, valid_n, nblk, imax_b)


def _nms_gathered(anch_g, dl_g, dr_g, valid_n, nblk, imax_b):
    B = anch_g.shape[0]

    def to_blocks(x):
        x = x.transpose(0, 2, 1).reshape(B, 4, nblk, _LANES).transpose(0, 2, 1, 3)
        return jnp.pad(x, ((0, 0), (0, 0), (1, 3), (0, 0)))

    out_l, out_r, cnts = pl.pallas_call(
        functools.partial(_nms_body, nblk, valid_n),
        grid=(B,),
        in_specs=[
            pl.BlockSpec((1, 8, _LANES), lambda i: (i, 0, 0)),
            pl.BlockSpec((1, nblk, 8, _LANES), lambda i: (i, 0, 0, 0)),
            pl.BlockSpec((1, nblk, 8, _LANES), lambda i: (i, 0, 0, 0)),
            pl.BlockSpec((1, nblk, 8, _LANES), lambda i: (i, 0, 0, 0)),
        ],
        out_specs=[
            pl.BlockSpec((1, _POST, 5), lambda i: (i, 0, 0)),
            pl.BlockSpec((1, _POST, 5), lambda i: (i, 0, 0)),
            pl.BlockSpec((1, 8, _LANES), lambda i: (i, 0, 0)),
        ],
        out_shape=[
            jax.ShapeDtypeStruct((B, _POST, 5), jnp.float32),
            jax.ShapeDtypeStruct((B, _POST, 5), jnp.float32),
            jax.ShapeDtypeStruct((B, 8, _LANES), jnp.int32),
        ],
        scratch_shapes=[
            pltpu.VMEM((nblk, 8, _LANES), jnp.float32),
            pltpu.VMEM((nblk, 8, _LANES), jnp.float32),
            pltpu.VMEM((nblk, _LANES), jnp.float32),
            pltpu.VMEM((nblk, _LANES), jnp.float32),
            pltpu.VMEM((3 * _LANES, 8), jnp.float32),
            pltpu.VMEM((3 * _LANES, 8), jnp.float32),
            pltpu.SMEM((1,), jnp.int32),
        ],
    )(imax_b, to_blocks(anch_g), to_blocks(dl_g), to_blocks(dr_g))
    return out_l, out_r, cnts


_FAST_N = 1024
_SR = 512
_NSC = 65472


_NEX = 16
_CR = _SR // 8


def _topk_body(sc_ref, out_ref, ok_ref):
    hp = jax.lax.Precision.HIGHEST
    dot = functools.partial(lax.dot_general, precision=hp)
    mm = lambda a, b: dot(a, b, (((1,), (0,)), ((), ())))
    x = sc_ref[0]
    liota = lax.broadcasted_iota(jnp.int32, (1, _LANES), 1)
    rowbase = (lax.broadcasted_iota(jnp.int32, (_SR, 1), 0)
               * _LANES).astype(jnp.float32)

    vcols, icols = [], []
    for _ in range(_NEX):
        m = jnp.max(x, axis=1, keepdims=True)
        eq = x == m
        lane = jnp.min(jnp.where(eq, liota, _LANES), axis=1, keepdims=True)
        fo = liota == lane
        vcols.append(m)
        icols.append(rowbase + lane.astype(jnp.float32))
        x = jnp.where(fo, -1.0, x)
    vals16 = jnp.concatenate(vcols, axis=1)
    idx16 = jnp.concatenate(icols, axis=1)
    eb = jnp.max(vals16[:, _NEX - 1:_NEX])

    kv = jnp.zeros((_CR, _LANES), jnp.float32)
    iv = jnp.zeros((_CR, _LANES), jnp.float32)
    r64 = lax.broadcasted_iota(jnp.int32, (_CR, _SR), 0)
    c64 = lax.broadcasted_iota(jnp.int32, (_CR, _SR), 1)
    t16 = lax.broadcasted_iota(jnp.int32, (_NEX, _LANES), 0)
    m16 = lax.broadcasted_iota(jnp.int32, (_NEX, _LANES), 1)
    for a in range(8):
        sel_a = jnp.where(c64 == 8 * r64 + a, 1.0, 0.0)
        spread_a = jnp.where(m16 == 16 * a + t16, 1.0, 0.0)
        kv = kv + mm(mm(sel_a, vals16), spread_a)
        iv = iv + mm(mm(sel_a, idx16), spread_a)

    riota = lax.broadcasted_iota(jnp.int32, (_CR, 1), 0)

    def rowroll(z, m_):
        return jnp.concatenate([z[m_:, :], z[:m_, :]], axis=0)

    def laneroll(z, d_):
        return jnp.concatenate([z[:, d_:], z[:, :d_]], axis=1)

    for st in range(1, 14):
        for d in [1 << b for b in range(st - 1, -1, -1)]:
            if d >= _LANES:
                mr = d >> 7
                is_lo = (riota & mr) == 0
                kp = jnp.where(is_lo, rowroll(kv, mr), rowroll(kv, _CR - mr))
                ip = jnp.where(is_lo, rowroll(iv, mr), rowroll(iv, _CR - mr))
            else:
                is_lo = (liota & d) == 0
                kp = jnp.where(is_lo, laneroll(kv, d), laneroll(kv, _LANES - d))
                ip = jnp.where(is_lo, laneroll(iv, d), laneroll(iv, _LANES - d))
            sbit = 1 << st
            if sbit >= _LANES:
                dirup = (riota & (sbit >> 7)) == 0
            else:
                dirup = (liota & sbit) == 0
            before = (kv > kp) | ((kv == kp) & (iv < ip))
            keep = before == (is_lo == dirup)
            kv = jnp.where(keep, kv, kp)
            iv = jnp.where(keep, iv, ip)

    kth = kv[7:8, _LANES - 1:_LANES]
    ok = (kth > eb).astype(jnp.int32)
    out_ref[0] = iv[0:8, :].astype(jnp.int32)
    ok_ref[0] = jnp.broadcast_to(ok, (8, _LANES))


def _topk_sorted(sf):
    B = sf.shape[0]
    sp = jnp.pad(sf, ((0, 0), (0, _SR * _LANES - _NSC)),
                 constant_values=-1.0).reshape(B, _SR, _LANES)
    out, ok = pl.pallas_call(
        _topk_body,
        grid=(B,),
        in_specs=[pl.BlockSpec((1, _SR, _LANES), lambda i: (i, 0, 0))],
        out_specs=[
            pl.BlockSpec((1, 8, _LANES), lambda i: (i, 0, 0)),
            pl.BlockSpec((1, 8, _LANES), lambda i: (i, 0, 0)),
        ],
        out_shape=[
            jax.ShapeDtypeStruct((B, 8, _LANES), jnp.int32),
            jax.ShapeDtypeStruct((B, 8, _LANES), jnp.int32),
        ],
    )(sp)
    return out.reshape(B, 8 * _LANES), ok[:, 0, 0]


def _sc_gather(deltas_flat, anchors, idx_flat, idx_anch):
    n = idx_flat.shape[0]
    nw = 32
    bpw = n // nw
    mesh = plsc.VectorSubcoreMesh(core_axis_name="c", subcore_axis_name="s")

    @functools.partial(
        pl.kernel, mesh=mesh,
        compiler_params=pltpu.CompilerParams(use_tc_tiling_on_sc=False),
        out_type=[
            jax.ShapeDtypeStruct((n, 8), jnp.float32),
            jax.ShapeDtypeStruct((n, 8), jnp.float32),
        ],
        scratch_types=[
            pltpu.VMEM((bpw,), jnp.int32),
            pltpu.VMEM((bpw,), jnp.int32),
            pltpu.VMEM((bpw, 8), jnp.float32),
            pltpu.VMEM((bpw, 8), jnp.float32),
            pltpu.SemaphoreType.DMA,
            pltpu.SemaphoreType.DMA,
        ])
    def k(d_hbm, a_hbm, if_hbm, ia_hbm, out_d, out_a,
          if_v, ia_v, drows_v, arows_v, semd, sema):
        wid = lax.axis_index("s") * 2 + lax.axis_index("c")
        base = wid * bpw
        pltpu.sync_copy(if_hbm.at[pl.ds(base, bpw)], if_v)
        pltpu.sync_copy(ia_hbm.at[pl.ds(base, bpw)], ia_v)
        cd = pltpu.async_copy(d_hbm.at[if_v], drows_v, semd)
        ca = pltpu.async_copy(a_hbm.at[ia_v], arows_v, sema)
        cd.wait()
        ca.wait()
        pltpu.sync_copy(drows_v, out_d.at[pl.ds(base, bpw)])
        pltpu.sync_copy(arows_v, out_a.at[pl.ds(base, bpw)])

    return k(deltas_flat, anchors, idx_flat, idx_anch)


def kernel(scores, bbox_deltas, im_info):
    B = scores.shape[0]
    sf = scores[:, :, 1]
    imax = jnp.stack([im_info[:, 1] - 1.0, im_info[:, 0] - 1.0], axis=1)
    imax_b = jnp.broadcast_to(
        jnp.pad(imax, ((0, 0), (0, 6)))[:, :, None], (B, 8, _LANES))

    ord_fast, ok_topk = _topk_sorted(sf)
    boff = (jnp.arange(B, dtype=jnp.int32) * _NSC)[:, None]
    deltas8 = jnp.pad(bbox_deltas, ((0, 0), (0, 0), (0, 2))).reshape(
        B * _NSC, 8)
    anch8 = jnp.asarray(np.pad(_ANCHORS, ((0, 0), (0, 4))))
    d_g, a_g = _sc_gather(deltas8, anch8,
                          (ord_fast + boff).reshape(-1),
                          ord_fast.reshape(-1))
    anch_g = a_g[:, :4].reshape(B, _FAST_N, 4)
    dlf_g = d_g[:, :4].reshape(B, _FAST_N, 4)
    drf_g = jnp.stack([d_g[:, 4], d_g[:, 1], d_g[:, 5], d_g[:, 3]],
                      axis=-1).reshape(B, _FAST_N, 4)
    fl, fr, fc = _nms_gathered(anch_g, dlf_g, drf_g, _FAST_N,
                               _FAST_N // _LANES, imax_b)

    def full_path():
        dl = bbox_deltas[..., :4]
        dr = jnp.stack([bbox_deltas[..., 4], bbox_deltas[..., 1],
                        bbox_deltas[..., 5], bbox_deltas[..., 3]], axis=-1)
        order = jnp.argsort(-sf, axis=1)[:, :_PRE]
        order = jnp.pad(order, ((0, 0), (0, _PAD_N - _PRE)))
        ol, og, _ = _nms_pipeline(order, _PRE, _NB, dl, dr, imax_b)
        return ol, og

    good = jnp.all(fc[:, 0, 0] >= _POST) & jnp.all(ok_topk == 1)
    return lax.cond(good, lambda: (fl, fr), full_path)
